# baseline (speedup 1.0000x reference)
import functools

import jax
import jax.numpy as jnp
from jax import lax
from jax.experimental import pallas as pl
from jax.experimental.pallas import tpu as pltpu
from jax.experimental.pallas import tpu_sc as plsc

_N = 16384
_C = 1000
_NS = 16
_NW = _NS
_RPW = _N // _NW
_HALF = _RPW // 2
_LANES = 16
_TILE_W = 128
_CHUNK = 128


def _sc_body(vt_hbm, idx_hbm, out_hbm, idx_v, lines_v, out_v,
             sem0, sem1, sem2, sem3):
    base = lax.axis_index("s") * _RPW
    sems = (sem0, sem1, sem2, sem3)

    pltpu.sync_copy(idx_hbm.at[pl.ds(base, _RPW)], idx_v)

    lane = lax.iota(jnp.int32, _LANES)
    for p in range(2):
        pbase = base + p * _HALF
        poff = p * _HALF

        copies = []
        for c in range(_HALF // _CHUNK):
            win = pl.multiple_of(pbase + c * _CHUNK, _TILE_W)
            copies.append(pltpu.async_copy(
                vt_hbm.at[idx_v.at[pl.ds(poff + c * _CHUNK, _CHUNK)],
                          pl.ds(win, _TILE_W)],
                lines_v.at[pl.ds(c * _CHUNK, _CHUNK), :],
                sems[c],
            ))

        for c, cp in enumerate(copies):
            cp.wait()
            for k in range(_CHUNK // _LANES):
                o = lane + (c * _CHUNK + k * _LANES)
                col = jnp.bitwise_and(o + poff, _TILE_W - 1)
                out_v[pl.ds(poff + c * _CHUNK + k * _LANES, _LANES)] = (
                    plsc.load_gather(lines_v, [o, col]))

    pltpu.sync_copy(out_v, out_hbm.at[pl.ds(base, _RPW)])


@jax.jit
def kernel(value, index):
    mesh = plsc.VectorSubcoreMesh(
        core_axis_name="c", subcore_axis_name="s", num_cores=1)
    run = functools.partial(
        pl.kernel,
        out_type=jax.ShapeDtypeStruct((_N,), jnp.float32),
        mesh=mesh,
        compiler_params=pltpu.CompilerParams(needs_layout_passes=False),
        scratch_types=[
            pltpu.VMEM((_RPW,), jnp.int32),
            pltpu.VMEM((_HALF, _TILE_W), jnp.float32),
            pltpu.VMEM((_RPW,), jnp.float32),
            pltpu.SemaphoreType.DMA,
            pltpu.SemaphoreType.DMA,
            pltpu.SemaphoreType.DMA,
            pltpu.SemaphoreType.DMA,
        ],
    )(_sc_body)
    flat = run(value.T, index.reshape(_N).astype(jnp.int32))
    return flat.reshape(_N, 1)


# speedup vs baseline: 1.1448x; 1.1448x over previous
import functools

import jax
import jax.numpy as jnp
from jax import lax
from jax.experimental import pallas as pl
from jax.experimental.pallas import tpu as pltpu
from jax.experimental.pallas import tpu_sc as plsc

_N = 16384
_C = 1000
_NC = 2
_NS = 16
_NW = _NC * _NS
_RPW = _N // _NW
_LANES = 16
_TILE_W = 128
_CHUNK = 128


def _sc_body(vt_hbm, idx_hbm, out_hbm, idx_v, lines_v, out_v,
             sem0, sem1, sem2, sem3):
    wid = lax.axis_index("s") * _NC + lax.axis_index("c")
    base = wid * _RPW
    sems = (sem0, sem1, sem2, sem3)

    pltpu.sync_copy(idx_hbm.at[pl.ds(base, _RPW)], idx_v)

    copies = []
    for c in range(_RPW // _CHUNK):
        win = pl.multiple_of(base + c * _CHUNK, _TILE_W)
        copies.append(pltpu.async_copy(
            vt_hbm.at[idx_v.at[pl.ds(c * _CHUNK, _CHUNK)],
                      pl.ds(win, _TILE_W)],
            lines_v.at[pl.ds(c * _CHUNK, _CHUNK), :],
            sems[c],
        ))

    lane = lax.iota(jnp.int32, _LANES)
    for c, cp in enumerate(copies):
        cp.wait()
        for k in range(_CHUNK // _LANES):
            o = lane + (c * _CHUNK + k * _LANES)
            col = jnp.bitwise_and(o, _TILE_W - 1)
            out_v[pl.ds(c * _CHUNK + k * _LANES, _LANES)] = (
                plsc.load_gather(lines_v, [o, col]))

    pltpu.sync_copy(out_v, out_hbm.at[pl.ds(base, _RPW)])


@jax.jit
def kernel(value, index):
    mesh = plsc.VectorSubcoreMesh(core_axis_name="c", subcore_axis_name="s")
    run = functools.partial(
        pl.kernel,
        out_type=jax.ShapeDtypeStruct((_N,), jnp.float32),
        mesh=mesh,
        compiler_params=pltpu.CompilerParams(needs_layout_passes=False),
        scratch_types=[
            pltpu.VMEM((_RPW,), jnp.int32),
            pltpu.VMEM((_RPW, _TILE_W), jnp.float32),
            pltpu.VMEM((_RPW,), jnp.float32),
            pltpu.SemaphoreType.DMA,
            pltpu.SemaphoreType.DMA,
            pltpu.SemaphoreType.DMA,
            pltpu.SemaphoreType.DMA,
        ],
    )(_sc_body)
    flat = run(value.T, index.reshape(_N).astype(jnp.int32))
    return flat.reshape(_N, 1)
